# Initial kernel scaffold; baseline (speedup 1.0000x reference)
import functools

import jax
import jax.numpy as jnp
from jax import lax
from jax.experimental import pallas as pl
from jax.experimental.pallas import tpu as pltpu
from jax.experimental.pallas import tpu_sc as plsc

N_NODES = 10000
N_EDGES = 320000
DIM = 128

NUM_CORES = 2
NUM_SUBCORES = 16
NUM_WORKERS = NUM_CORES * NUM_SUBCORES
CHUNK = 128
CHUNKS_PER_WORKER = -(-N_EDGES // (NUM_WORKERS * CHUNK))
E_PAD = NUM_WORKERS * CHUNKS_PER_WORKER * CHUNK
ROWS_PER_TILE = N_NODES // NUM_SUBCORES


def _matmul_body(x_ref, w_ref, o_ref):
    o_ref[...] = jnp.dot(x_ref[...], w_ref[...],
                         preferred_element_type=jnp.float32)


def _support_matmul(x, W):
    blk = 400
    grid = N_NODES // blk
    return pl.pallas_call(
        _matmul_body,
        grid=(grid,),
        in_specs=[
            pl.BlockSpec((blk, DIM), lambda i: (i, 0)),
            pl.BlockSpec((DIM, DIM), lambda i: (0, 0)),
        ],
        out_specs=pl.BlockSpec((blk, DIM), lambda i: (i, 0)),
        out_shape=jax.ShapeDtypeStruct((N_NODES, DIM), jnp.float32),
    )(x, W)


def _combine_body(p_ref, b_ref, o_ref):
    o_ref[...] = p_ref[0] + p_ref[1] + b_ref[...]


def _combine(partials, b2d):
    blk = 400
    grid = N_NODES // blk
    return pl.pallas_call(
        _combine_body,
        grid=(grid,),
        in_specs=[
            pl.BlockSpec((2, blk, DIM), lambda i: (0, i, 0)),
            pl.BlockSpec((1, DIM), lambda i: (0, 0)),
        ],
        out_specs=pl.BlockSpec((blk, DIM), lambda i: (i, 0)),
        out_shape=jax.ShapeDtypeStruct((N_NODES, DIM), jnp.float32),
    )(partials, b2d)


_MESH = plsc.VectorSubcoreMesh(core_axis_name="c", subcore_axis_name="s")


@functools.partial(
    pl.kernel,
    out_type=jax.ShapeDtypeStruct((NUM_CORES, N_NODES, DIM), jnp.float32),
    mesh=_MESH,
    scratch_types=[
        pltpu.MemoryRef((N_NODES, DIM), jnp.float32, pltpu.VMEM_SHARED),
        pltpu.VMEM((CHUNK,), jnp.int32),
        pltpu.VMEM((CHUNK,), jnp.int32),
        pltpu.VMEM((CHUNK,), jnp.float32),
        pltpu.VMEM((CHUNK, DIM), jnp.float32),
        pltpu.SemaphoreType.DMA,
    ],
)
def _sc_aggregate(support_hbm, src_hbm, dst_hbm, w_hbm, out_hbm,
                  acc, src_v, dst_v, w_v, rows_v, sem):
    cid = lax.axis_index("c")
    sid = lax.axis_index("s")
    wid = cid * NUM_SUBCORES + sid

    @pl.loop(0, CHUNK)
    def _zero_row(j):
        for k in range(DIM // 16):
            rows_v[j, pl.ds(k * 16, 16)] = jnp.zeros((16,), jnp.float32)

    base = sid * ROWS_PER_TILE
    full = ROWS_PER_TILE // CHUNK
    rem = ROWS_PER_TILE - full * CHUNK
    for i in range(full):
        pltpu.sync_copy(rows_v, acc.at[pl.ds(base + i * CHUNK, CHUNK)])
    if rem:
        pltpu.sync_copy(rows_v.at[pl.ds(0, rem)],
                        acc.at[pl.ds(base + full * CHUNK, rem)])

    plsc.subcore_barrier()

    @pl.loop(0, CHUNKS_PER_WORKER)
    def _chunk(c):
        pltpu.sync_copy(src_hbm.at[wid, c], src_v)
        pltpu.sync_copy(dst_hbm.at[wid, c], dst_v)
        pltpu.sync_copy(w_hbm.at[wid, c], w_v)
        pltpu.async_copy(support_hbm.at[src_v], rows_v, sem).wait()

        @pl.loop(0, CHUNK)
        def _edge(j):
            wj = plsc.load_gather(w_v, [jnp.full((16,), j, jnp.int32)])
            for k in range(DIM // 16):
                sl = (j, pl.ds(k * 16, 16))
                rows_v[sl] = rows_v[sl] * wj

        pltpu.sync_copy(rows_v, acc.at[dst_v], add=True)

    plsc.subcore_barrier()

    for i in range(full):
        pltpu.sync_copy(acc.at[pl.ds(base + i * CHUNK, CHUNK)],
                        out_hbm.at[cid, pl.ds(base + i * CHUNK, CHUNK)])
    if rem:
        pltpu.sync_copy(acc.at[pl.ds(base + full * CHUNK, rem)],
                        out_hbm.at[cid, pl.ds(base + full * CHUNK, rem)])


def kernel(input, edge_index, edge_weight, W, b):
    support = _support_matmul(input, W)

    pad = E_PAD - N_EDGES
    src = jnp.concatenate(
        [edge_index[1].astype(jnp.int32), jnp.zeros((pad,), jnp.int32)])
    dst = jnp.concatenate(
        [edge_index[0].astype(jnp.int32), jnp.zeros((pad,), jnp.int32)])
    w = jnp.concatenate(
        [edge_weight.astype(jnp.float32), jnp.zeros((pad,), jnp.float32)])
    src3 = src.reshape(NUM_WORKERS, CHUNKS_PER_WORKER, CHUNK)
    dst3 = dst.reshape(NUM_WORKERS, CHUNKS_PER_WORKER, CHUNK)
    w3 = w.reshape(NUM_WORKERS, CHUNKS_PER_WORKER, CHUNK)

    partials = _sc_aggregate(support, src3, dst3, w3)
    return _combine(partials, b.reshape(1, DIM))


# speedup vs baseline: 3.3498x; 3.3498x over previous
import functools

import jax
import jax.numpy as jnp
from jax import lax
from jax.experimental import pallas as pl
from jax.experimental.pallas import tpu as pltpu
from jax.experimental.pallas import tpu_sc as plsc

N_NODES = 10000
N_EDGES = 320000
DIM = 128

NUM_CORES = 2
NUM_SUBCORES = 16
NUM_WORKERS = NUM_CORES * NUM_SUBCORES
CHUNK = 128
CHUNKS_PER_WORKER = -(-N_EDGES // (NUM_WORKERS * CHUNK))
E_PAD = NUM_WORKERS * CHUNKS_PER_WORKER * CHUNK
N_PAD = 10240
ROWS_PER_TILE = N_PAD // NUM_SUBCORES


def _matmul_body(x_ref, w_ref, o_ref):
    o_ref[...] = jnp.dot(x_ref[...], w_ref[...],
                         preferred_element_type=jnp.float32)


def _support_matmul(x, W):
    blk = 400
    grid = N_NODES // blk
    return pl.pallas_call(
        _matmul_body,
        grid=(grid,),
        in_specs=[
            pl.BlockSpec((blk, DIM), lambda i: (i, 0)),
            pl.BlockSpec((DIM, DIM), lambda i: (0, 0)),
        ],
        out_specs=pl.BlockSpec((blk, DIM), lambda i: (i, 0)),
        out_shape=jax.ShapeDtypeStruct((N_NODES, DIM), jnp.float32),
    )(x, W)


def _combine_body(p_ref, b_ref, o_ref):
    o_ref[...] = p_ref[0] + p_ref[1] + b_ref[...]


def _combine(partials, b2d):
    blk = 400
    grid = N_NODES // blk
    return pl.pallas_call(
        _combine_body,
        grid=(grid,),
        in_specs=[
            pl.BlockSpec((2, blk, DIM), lambda i: (0, i, 0)),
            pl.BlockSpec((1, DIM), lambda i: (0, 0)),
        ],
        out_specs=pl.BlockSpec((blk, DIM), lambda i: (i, 0)),
        out_shape=jax.ShapeDtypeStruct((N_NODES, DIM), jnp.float32),
    )(partials, b2d)


_MESH = plsc.VectorSubcoreMesh(core_axis_name="c", subcore_axis_name="s")


@functools.partial(
    pl.kernel,
    out_type=jax.ShapeDtypeStruct((NUM_CORES, N_PAD, DIM), jnp.float32),
    mesh=_MESH,
    scratch_types=[
        pltpu.VMEM_SHARED((N_PAD, DIM), jnp.float32),
        pltpu.VMEM((CHUNK,), jnp.int32),
        pltpu.VMEM((CHUNK,), jnp.int32),
        pltpu.VMEM((CHUNK,), jnp.float32),
        pltpu.VMEM((CHUNK, DIM), jnp.float32),
        pltpu.SemaphoreType.DMA,
    ],
    compiler_params=pltpu.CompilerParams(needs_layout_passes=False),
)
def _sc_aggregate(support_hbm, src_hbm, dst_hbm, w_hbm, out_hbm,
                  acc, src_v, dst_v, w_v, rows_v, sem):
    cid = lax.axis_index("c")
    sid = lax.axis_index("s")
    wid = cid * NUM_SUBCORES + sid

    @pl.loop(0, CHUNK)
    def _zero_row(j):
        for k in range(DIM // 16):
            rows_v[j, pl.ds(k * 16, 16)] = jnp.zeros((16,), jnp.float32)

    base = sid * ROWS_PER_TILE
    full = ROWS_PER_TILE // CHUNK
    for i in range(full):
        pltpu.sync_copy(rows_v, acc.at[pl.ds(base + i * CHUNK, CHUNK)])

    plsc.subcore_barrier()

    @pl.loop(0, CHUNKS_PER_WORKER)
    def _chunk(c):
        pltpu.sync_copy(src_hbm.at[wid, c], src_v)
        pltpu.sync_copy(dst_hbm.at[wid, c], dst_v)
        pltpu.sync_copy(w_hbm.at[wid, c], w_v)
        pltpu.async_copy(support_hbm.at[src_v], rows_v, sem).wait()

        @pl.loop(0, CHUNK)
        def _edge(j):
            wj = plsc.load_gather(w_v, [jnp.full((16,), j, jnp.int32)])
            for k in range(DIM // 16):
                sl = (j, pl.ds(k * 16, 16))
                rows_v[sl] = rows_v[sl] * wj

        pltpu.sync_copy(rows_v, acc.at[dst_v], add=True)

    plsc.subcore_barrier()

    for i in range(full):
        pltpu.sync_copy(acc.at[pl.ds(base + i * CHUNK, CHUNK)],
                        out_hbm.at[cid, pl.ds(base + i * CHUNK, CHUNK)])


def kernel(input, edge_index, edge_weight, W, b):
    support = _support_matmul(input, W)

    pad = E_PAD - N_EDGES
    src = jnp.concatenate(
        [edge_index[1].astype(jnp.int32), jnp.zeros((pad,), jnp.int32)])
    dst = jnp.concatenate(
        [edge_index[0].astype(jnp.int32), jnp.zeros((pad,), jnp.int32)])
    w = jnp.concatenate(
        [edge_weight.astype(jnp.float32), jnp.zeros((pad,), jnp.float32)])
    src3 = src.reshape(NUM_WORKERS, CHUNKS_PER_WORKER, CHUNK)
    dst3 = dst.reshape(NUM_WORKERS, CHUNKS_PER_WORKER, CHUNK)
    w3 = w.reshape(NUM_WORKERS, CHUNKS_PER_WORKER, CHUNK)

    partials = _sc_aggregate(support, src3, dst3, w3)
    return _combine(partials, b.reshape(1, DIM))


# speedup vs baseline: 3.8443x; 1.1476x over previous
import functools

import jax
import jax.numpy as jnp
from jax import lax
from jax.experimental import pallas as pl
from jax.experimental.pallas import tpu as pltpu
from jax.experimental.pallas import tpu_sc as plsc

N_NODES = 10000
N_EDGES = 320000
DIM = 128

NUM_CORES = 2
NUM_SUBCORES = 16
NUM_WORKERS = NUM_CORES * NUM_SUBCORES
CHUNK = 128
GROUP = 8
CPW = 80
E_PAD = NUM_WORKERS * CPW * CHUNK
N_PAD = 10240
ROWS_PER_TILE = N_PAD // NUM_SUBCORES
NROW = 2
NIDX = 8


def _matmul_body(x_ref, w_ref, o_ref):
    o_ref[...] = jnp.dot(x_ref[...], w_ref[...],
                         preferred_element_type=jnp.float32)


def _support_matmul(x, W):
    blk = 400
    grid = N_NODES // blk
    return pl.pallas_call(
        _matmul_body,
        grid=(grid,),
        in_specs=[
            pl.BlockSpec((blk, DIM), lambda i: (i, 0)),
            pl.BlockSpec((DIM, DIM), lambda i: (0, 0)),
        ],
        out_specs=pl.BlockSpec((blk, DIM), lambda i: (i, 0)),
        out_shape=jax.ShapeDtypeStruct((N_NODES, DIM), jnp.float32),
    )(x, W)


def _combine_body(p_ref, b_ref, o_ref):
    o_ref[...] = p_ref[0] + p_ref[1] + b_ref[...]


def _combine(partials, b2d):
    blk = 400
    grid = N_NODES // blk
    return pl.pallas_call(
        _combine_body,
        grid=(grid,),
        in_specs=[
            pl.BlockSpec((2, blk, DIM), lambda i: (0, i, 0)),
            pl.BlockSpec((1, DIM), lambda i: (0, 0)),
        ],
        out_specs=pl.BlockSpec((blk, DIM), lambda i: (i, 0)),
        out_shape=jax.ShapeDtypeStruct((N_NODES, DIM), jnp.float32),
    )(partials, b2d)


_MESH = plsc.VectorSubcoreMesh(core_axis_name="c", subcore_axis_name="s")


@functools.partial(
    pl.kernel,
    out_type=jax.ShapeDtypeStruct((NUM_CORES, N_PAD, DIM), jnp.float32),
    mesh=_MESH,
    scratch_types=[
        pltpu.VMEM_SHARED((N_PAD, DIM), jnp.float32),
        pltpu.VMEM((NIDX, 3, CHUNK), jnp.int32),
        pltpu.VMEM((NROW, CHUNK, DIM), jnp.float32),
        pltpu.SemaphoreType.DMA((NIDX,)),
        pltpu.SemaphoreType.DMA((NROW,)),
        pltpu.SemaphoreType.DMA((NROW,)),
    ],
    compiler_params=pltpu.CompilerParams(needs_layout_passes=False),
)
def _sc_aggregate(support_hbm, edges_hbm, out_hbm,
                  acc, idx_v, rows_v, isem, gsem, ssem):
    cid = lax.axis_index("c")
    sid = lax.axis_index("s")
    wid = cid * NUM_SUBCORES + sid

    @pl.loop(0, CHUNK)
    def _zero_row(j):
        for k in range(DIM // 16):
            rows_v[0, j, pl.ds(k * 16, 16)] = jnp.zeros((16,), jnp.float32)

    base = sid * ROWS_PER_TILE
    for i in range(ROWS_PER_TILE // CHUNK):
        pltpu.sync_copy(rows_v.at[0], acc.at[pl.ds(base + i * CHUNK, CHUNK)])

    plsc.subcore_barrier()

    def _idx_copy(j, q):
        return pltpu.async_copy(edges_hbm.at[wid, j], idx_v.at[q],
                                isem.at[q])

    def _gather(j, q, b):
        del j
        return pltpu.async_copy(support_hbm.at[idx_v.at[q, 0]],
                                rows_v.at[b], gsem.at[b])

    def _scatter(q, b):
        return pltpu.async_copy(rows_v.at[b], acc.at[idx_v.at[q, 1]],
                                ssem.at[b], add=True)

    for j in range(2):
        _idx_copy(j, j)
    pltpu.make_async_copy(edges_hbm.at[wid, 0], idx_v.at[0],
                          isem.at[0]).wait()
    _gather(0, 0, 0)

    @pl.loop(0, CPW // GROUP)
    def _grp(g):
        c0 = g * GROUP
        for k in range(GROUP):
            i = c0 + k
            b = k % NROW
            q = k
            b1 = (k + 1) % NROW
            q1 = (k + 1) % NIDX
            q2 = (k + 2) % NIDX

            @pl.when(i + 2 < CPW)
            def _():
                _idx_copy(i + 2, q2)

            @pl.when(i + 1 < CPW)
            def _():
                @pl.when(i + 1 >= NROW)
                def _():
                    pltpu.make_async_copy(
                        rows_v.at[b1], acc.at[idx_v.at[q1, 1]],
                        ssem.at[b1]).wait()
                pltpu.make_async_copy(edges_hbm.at[wid, i + 1],
                                      idx_v.at[q1], isem.at[q1]).wait()
                _gather(i + 1, q1, b1)

            pltpu.make_async_copy(support_hbm.at[idx_v.at[q, 0]],
                                  rows_v.at[b], gsem.at[b]).wait()

            @plsc.parallel_loop(0, CHUNK, unroll=2)
            def _edge(j):
                wbits = plsc.load_gather(idx_v.at[q, 2],
                                         [jnp.full((16,), j, jnp.int32)])
                wj = plsc.bitcast(wbits, jnp.float32)
                for kk in range(DIM // 16):
                    sl = (b, j, pl.ds(kk * 16, 16))
                    rows_v[sl] = rows_v[sl] * wj

            _scatter(q, b)

    for k in range(GROUP - NROW, GROUP):
        pltpu.make_async_copy(rows_v.at[k % NROW], acc.at[idx_v.at[k, 1]],
                              ssem.at[k % NROW]).wait()

    plsc.subcore_barrier()

    for i in range(ROWS_PER_TILE // CHUNK):
        pltpu.sync_copy(acc.at[pl.ds(base + i * CHUNK, CHUNK)],
                        out_hbm.at[cid, pl.ds(base + i * CHUNK, CHUNK)])


def kernel(input, edge_index, edge_weight, W, b):
    support = _support_matmul(input, W)

    pad = E_PAD - N_EDGES
    src = jnp.concatenate(
        [edge_index[1].astype(jnp.int32), jnp.zeros((pad,), jnp.int32)])
    dst = jnp.concatenate(
        [edge_index[0].astype(jnp.int32), jnp.zeros((pad,), jnp.int32)])
    wbits = jnp.concatenate(
        [lax.bitcast_convert_type(edge_weight.astype(jnp.float32), jnp.int32),
         jnp.zeros((pad,), jnp.int32)])
    edges = jnp.stack(
        [src.reshape(NUM_WORKERS, CPW, CHUNK),
         dst.reshape(NUM_WORKERS, CPW, CHUNK),
         wbits.reshape(NUM_WORKERS, CPW, CHUNK)], axis=2)

    partials = _sc_aggregate(support, edges)
    return _combine(partials, b.reshape(1, DIM))
